# baseline (speedup 1.0000x reference)
import functools

import jax
import jax.numpy as jnp
from jax import lax
from jax.experimental import pallas as pl
from jax.experimental.pallas import tpu as pltpu
from jax.experimental.pallas import tpu_sc as plsc

_B, _T, _D = 1024, 50, 300
_CHUNK = 2


def _sc_copy(v_hbm, o_hbm, buf0, buf1, sem_in, sem_out, nc):
    wid = lax.axis_index("s") * nc + lax.axis_index("c")
    rows_per_w = _B // (nc * 16)
    nchunks = rows_per_w // _CHUNK
    base = wid * rows_per_w
    bufs = (buf0, buf1)

    ins = [
        pltpu.make_async_copy(
            v_hbm.at[pl.ds(base + i * _CHUNK, _CHUNK)], bufs[i % 2], sem_in.at[i % 2]
        )
        for i in range(nchunks)
    ]
    outs = [
        pltpu.make_async_copy(
            bufs[i % 2], o_hbm.at[pl.ds(base + i * _CHUNK, _CHUNK)], sem_out.at[i % 2]
        )
        for i in range(nchunks)
    ]

    ins[0].start()
    for i in range(nchunks):
        ins[i].wait()
        outs[i].start()
        if i + 1 < nchunks:
            if i >= 1:
                outs[i - 1].wait()
            ins[i + 1].start()
    if nchunks >= 2:
        outs[nchunks - 2].wait()
    outs[nchunks - 1].wait()


def kernel(video, ques, attr, emb):
    del ques, attr, emb
    info = plsc.get_sparse_core_info()
    nc = info.num_cores
    mesh = plsc.VectorSubcoreMesh(core_axis_name="c", subcore_axis_name="s")
    k = functools.partial(
        pl.kernel,
        out_type=jax.ShapeDtypeStruct((_B, _T, _D), jnp.float32),
        mesh=mesh,
        scratch_types=[
            pltpu.VMEM((_CHUNK, _T, _D), jnp.float32),
            pltpu.VMEM((_CHUNK, _T, _D), jnp.float32),
            pltpu.SemaphoreType.DMA((2,)),
            pltpu.SemaphoreType.DMA((2,)),
        ],
    )(functools.partial(_sc_copy, nc=nc))
    return k(video)


# speedup vs baseline: 1.0028x; 1.0028x over previous
import functools

import jax
import jax.numpy as jnp
from jax import lax
from jax.experimental import pallas as pl
from jax.experimental.pallas import tpu as pltpu
from jax.experimental.pallas import tpu_sc as plsc

_B, _T, _D = 1024, 50, 300
_CHUNK = 1
_NBUF = 5
_AHEAD = 3


def _sc_copy(v_hbm, o_hbm, bufs, sem_in, sem_out, nc):
    wid = lax.axis_index("s") * nc + lax.axis_index("c")
    rows_per_w = _B // (nc * 16)
    n = rows_per_w // _CHUNK
    base = wid * rows_per_w

    ins = [
        pltpu.make_async_copy(
            v_hbm.at[pl.ds(base + i * _CHUNK, _CHUNK)],
            bufs[i % _NBUF],
            sem_in.at[i % _NBUF],
        )
        for i in range(n)
    ]
    outs = [
        pltpu.make_async_copy(
            bufs[i % _NBUF],
            o_hbm.at[pl.ds(base + i * _CHUNK, _CHUNK)],
            sem_out.at[i % _NBUF],
        )
        for i in range(n)
    ]

    waited = [False] * n
    for i in range(_AHEAD):
        ins[i].start()
    for i in range(n):
        ins[i].wait()
        outs[i].start()
        j = i + _AHEAD
        if j < n:
            k = j - _NBUF
            if k >= 0:
                outs[k].wait()
                waited[k] = True
            ins[j].start()
    for i in range(n):
        if not waited[i]:
            outs[i].wait()


def kernel(video, ques, attr, emb):
    del ques, attr, emb
    info = plsc.get_sparse_core_info()
    nc = info.num_cores
    mesh = plsc.VectorSubcoreMesh(core_axis_name="c", subcore_axis_name="s")

    def body(v_hbm, o_hbm, b0, b1, b2, b3, b4, sem_in, sem_out):
        _sc_copy(v_hbm, o_hbm, (b0, b1, b2, b3, b4), sem_in, sem_out, nc)

    k = pl.kernel(
        body,
        out_type=jax.ShapeDtypeStruct((_B, _T, _D), jnp.float32),
        mesh=mesh,
        scratch_types=[
            pltpu.VMEM((_CHUNK, _T, _D), jnp.float32),
            pltpu.VMEM((_CHUNK, _T, _D), jnp.float32),
            pltpu.VMEM((_CHUNK, _T, _D), jnp.float32),
            pltpu.VMEM((_CHUNK, _T, _D), jnp.float32),
            pltpu.VMEM((_CHUNK, _T, _D), jnp.float32),
            pltpu.SemaphoreType.DMA((_NBUF,)),
            pltpu.SemaphoreType.DMA((_NBUF,)),
        ],
    )
    return k(video)


# speedup vs baseline: 1.1142x; 1.1111x over previous
import jax
import jax.numpy as jnp
from jax.experimental import pallas as pl
from jax.experimental.pallas import tpu as pltpu

_BLOCK_B = 64


def _copy_block(v_ref, o_ref):
    o_ref[...] = v_ref[...]


def kernel(video, ques, attr, emb):
    del ques, attr, emb
    b, t, d = video.shape
    out = pl.pallas_call(
        _copy_block,
        grid=(b // _BLOCK_B,),
        in_specs=[pl.BlockSpec((_BLOCK_B, t, d), lambda i: (i, 0, 0))],
        out_specs=pl.BlockSpec((_BLOCK_B, t, d), lambda i: (i, 0, 0)),
        out_shape=jax.ShapeDtypeStruct(video.shape, video.dtype),
        compiler_params=pltpu.CompilerParams(
            dimension_semantics=("parallel",),
        ),
    )(video)
    return out
